# Initial kernel scaffold; baseline (speedup 1.0000x reference)
import jax
import jax.numpy as jnp
from jax.experimental import pallas as pl


def kernel(z, W):
    raise NotImplementedError("write your pallas kernel here")


# speedup vs baseline: 1.3140x; 1.3140x over previous
import functools

import jax
import jax.numpy as jnp
from jax import lax
from jax.experimental import pallas as pl
from jax.experimental.pallas import tpu as pltpu
from jax.experimental.pallas import tpu_sc as plsc

_K = 1024
_D = 64
_N = 36864
_R = 1024
_G = _N // _R
_COST = 0.25

_NW = 32
_BPW = _N // _NW
_CH = 128
_NCH = _BPW // _CH


def _dist_body(z_ref, zsq_ref, w_ref, wsq_ref, codes_ref, loss_ref):
    z = z_ref[...]
    t = lax.dot_general(z, w_ref[...],
                        dimension_numbers=(((1,), (1,)), ((), ())),
                        preferred_element_type=jnp.float32)
    dist = (zsq_ref[...] + wsq_ref[...]) - 2.0 * t
    m = jnp.min(dist, axis=1, keepdims=True)
    ii = lax.broadcasted_iota(jnp.int32, dist.shape, 1)
    codes = jnp.min(jnp.where(dist == m, ii, jnp.int32(_K)), axis=1)
    codes_ref[0, 0, :] = codes

    @pl.when(pl.program_id(0) == 0)
    def _init():
        loss_ref[...] = jnp.zeros_like(loss_ref)

    loss_ref[...] += jnp.sum(m).reshape(1, 1)


_dist_call = pl.pallas_call(
    _dist_body,
    grid=(_G,),
    in_specs=[
        pl.BlockSpec((_R, _D), lambda i: (i, 0)),
        pl.BlockSpec((_R, 1), lambda i: (i, 0)),
        pl.BlockSpec((_K, _D), lambda i: (0, 0)),
        pl.BlockSpec((1, _K), lambda i: (0, 0)),
    ],
    out_specs=[
        pl.BlockSpec((1, 1, _R), lambda i: (i, 0, 0)),
        pl.BlockSpec((1, 1), lambda i: (0, 0)),
    ],
    out_shape=[
        jax.ShapeDtypeStruct((_G, 1, _R), jnp.int32),
        jax.ShapeDtypeStruct((1, 1), jnp.float32),
    ],
)


def _gather_body(w_hbm, codes_hbm, out_hbm, idx_v, rows_v, sem):
    wid = lax.axis_index("s") * 2 + lax.axis_index("c")
    pltpu.sync_copy(codes_hbm.at[wid], idx_v)
    copies = [
        pltpu.async_copy(w_hbm.at[idx_v.at[j]],
                         rows_v.at[pl.ds(j * _CH, _CH)], sem)
        for j in range(_NCH)
    ]
    for c in copies:
        c.wait()
    pltpu.sync_copy(rows_v, out_hbm.at[pl.ds(wid * _BPW, _BPW)])


def _gather_call(W, codes2d):
    run = functools.partial(
        pl.kernel,
        mesh=plsc.VectorSubcoreMesh(core_axis_name="c", subcore_axis_name="s"),
        out_type=jax.ShapeDtypeStruct((_N, _D), jnp.float32),
        scratch_types=[
            pltpu.VMEM((_NCH, _CH), jnp.int32),
            pltpu.VMEM((_BPW, _D), jnp.float32),
            pltpu.SemaphoreType.DMA,
        ],
        compiler_params=pltpu.CompilerParams(use_tc_tiling_on_sc=False),
    )(_gather_body)
    return run(W, codes2d)


def kernel(z, W):
    zf = z.reshape(_N, _D)
    zsq = jnp.sum(zf ** 2, axis=1, keepdims=True)
    wsq = jnp.sum(W ** 2, axis=1)[None, :]
    codes3d, loss_acc = _dist_call(zf, zsq, W, wsq)
    codes = codes3d.reshape(_N)
    z_q = _gather_call(W, codes.reshape(_NW, _NCH, _CH))
    vq_loss = (loss_acc[0, 0] / jnp.float32(_N * _D)) * jnp.float32(_COST)
    return (vq_loss, z_q.reshape(z.shape), codes.reshape(z.shape[0], -1))


# speedup vs baseline: 1.9218x; 1.4626x over previous
import functools

import jax
import jax.numpy as jnp
from jax import lax
from jax.experimental import pallas as pl
from jax.experimental.pallas import tpu as pltpu
from jax.experimental.pallas import tpu_sc as plsc

_K = 1024
_D = 64
_N = 36864
_R = 1024
_G = _N // _R
_COST = 0.25

_NW = 32
_BPW = _N // _NW
_CH = 128
_NCH = _BPW // _CH


def _dist_body(z_ref, zsq_ref, wm2_ref, wsq_ref, codes_ref, loss_ref):
    t2 = lax.dot_general(wm2_ref[...], z_ref[...],
                         dimension_numbers=(((1,), (1,)), ((), ())),
                         preferred_element_type=jnp.float32)
    dist = (zsq_ref[...] + wsq_ref[...]) + t2
    m = jnp.min(dist, axis=0, keepdims=True)
    ii = lax.broadcasted_iota(jnp.int32, dist.shape, 0).astype(jnp.float32)
    codes_f = jnp.min(jnp.where(dist == m, ii, jnp.float32(_K)), axis=0)
    codes_ref[0, 0, :] = codes_f.astype(jnp.int32)

    @pl.when(pl.program_id(0) == 0)
    def _init():
        loss_ref[...] = jnp.zeros_like(loss_ref)

    loss_ref[...] += jnp.sum(m).reshape(1, 1)


_dist_call = pl.pallas_call(
    _dist_body,
    grid=(_G,),
    in_specs=[
        pl.BlockSpec((_R, _D), lambda i: (i, 0)),
        pl.BlockSpec((1, _R), lambda i: (0, i)),
        pl.BlockSpec((_K, _D), lambda i: (0, 0)),
        pl.BlockSpec((_K, 1), lambda i: (0, 0)),
    ],
    out_specs=[
        pl.BlockSpec((1, 1, _R), lambda i: (i, 0, 0)),
        pl.BlockSpec((1, 1), lambda i: (0, 0)),
    ],
    out_shape=[
        jax.ShapeDtypeStruct((_G, 1, _R), jnp.int32),
        jax.ShapeDtypeStruct((1, 1), jnp.float32),
    ],
)


def _gather_body(w_hbm, codes_hbm, out_hbm, idx_v, rows_v, sem):
    wid = lax.axis_index("s") * 2 + lax.axis_index("c")
    pltpu.sync_copy(codes_hbm.at[wid], idx_v)
    copies = [
        pltpu.async_copy(w_hbm.at[idx_v.at[j]],
                         rows_v.at[pl.ds(j * _CH, _CH)], sem)
        for j in range(_NCH)
    ]
    for c in copies:
        c.wait()
    pltpu.sync_copy(rows_v, out_hbm.at[pl.ds(wid * _BPW, _BPW)])


def _gather_call(W, codes2d):
    run = functools.partial(
        pl.kernel,
        mesh=plsc.VectorSubcoreMesh(core_axis_name="c", subcore_axis_name="s"),
        out_type=jax.ShapeDtypeStruct((_N, _D), jnp.float32),
        scratch_types=[
            pltpu.VMEM((_NCH, _CH), jnp.int32),
            pltpu.VMEM((_BPW, _D), jnp.float32),
            pltpu.SemaphoreType.DMA,
        ],
        compiler_params=pltpu.CompilerParams(use_tc_tiling_on_sc=False),
    )(_gather_body)
    return run(W, codes2d)


def kernel(z, W):
    zf = z.reshape(_N, _D)
    zsq = jnp.sum(zf ** 2, axis=1)[None, :]
    wsq = jnp.sum(W ** 2, axis=1, keepdims=True)
    codes3d, loss_acc = _dist_call(zf, zsq, W * jnp.float32(-2.0), wsq)
    codes = codes3d.reshape(_N)
    z_q = _gather_call(W, codes.reshape(_NW, _NCH, _CH))
    vq_loss = (loss_acc[0, 0] / jnp.float32(_N * _D)) * jnp.float32(_COST)
    return (vq_loss, z_q.reshape(z.shape), codes.reshape(z.shape[0], -1))
